# baseline (speedup 1.0000x reference)
import functools

import jax
import jax.numpy as jnp
from jax import lax
from jax.experimental import pallas as pl
from jax.experimental.pallas import tpu as pltpu
from jax.experimental.pallas import tpu_sc as plsc

BATCH = 4096
SEQ = 200
N_TOKENS = 10
EMBED_DIM = 64
LANES = 16

_SC_INFO = plsc.get_sparse_core_info()
NUM_WORKERS = _SC_INFO.num_cores * _SC_INFO.num_subcores
BT = BATCH // NUM_WORKERS
JT = EMBED_DIM // 8
NBUF = 2
JCH = EMBED_DIM // LANES


@functools.partial(
    pl.kernel,
    out_type=jax.ShapeDtypeStruct((SEQ, JT, NUM_WORKERS, 8, BT), jnp.float32),
    mesh=plsc.VectorSubcoreMesh(core_axis_name="c", subcore_axis_name="s"),
    scratch_types=[
        pltpu.VMEM((SEQ * BT,), jnp.int32),
        pltpu.VMEM((NBUF, BT, EMBED_DIM), jnp.float32),
        pltpu.VMEM((NBUF, EMBED_DIM, BT), jnp.float32),
        pltpu.VMEM((N_TOKENS, EMBED_DIM), jnp.float32),
        pltpu.SemaphoreType.DMA,
        [pltpu.SemaphoreType.DMA] * NBUF,
        [pltpu.SemaphoreType.DMA] * NBUF,
    ],
    compiler_params=pltpu.CompilerParams(use_tc_tiling_on_sc=False,
                                         needs_layout_passes=False),
)
def _soft_embedding_sc(idx_hbm, table_hbm, learned_hbm, out_hbm,
                       idx_v, tbuf, obuf, learned_v, sem_i, sem_g, sem_w):
    wid = lax.axis_index("s") * _SC_INFO.num_cores + lax.axis_index("c")

    pltpu.sync_copy(learned_hbm, learned_v)
    pltpu.async_copy(idx_hbm.at[pl.ds(wid * SEQ * BT, SEQ * BT)],
                     idx_v, sem_i)
    jrows = [lax.iota(jnp.int32, LANES) + (c * LANES) for c in range(JCH)]
    pltpu.make_async_copy(idx_hbm.at[pl.ds(0, SEQ * BT)], idx_v, sem_i).wait()

    pltpu.async_copy(table_hbm.at[idx_v.at[pl.ds(0, BT)]],
                     tbuf.at[0], sem_g[0])

    def _write_waits(n):
        for jt in range(JT):
            pltpu.make_async_copy(
                obuf.at[n, pl.ds(jt * 8, 8)],
                out_hbm.at[0, jt, 0], sem_w[n]).wait()

    @pl.loop(0, SEQ, step=NBUF)
    def _(s):
        for n in range(NBUF):
            si = s + n
            nb = (n + 1) % NBUF

            @pl.when(si + 1 < SEQ)
            def _():
                pltpu.async_copy(
                    table_hbm.at[idx_v.at[pl.ds((si + 1) * BT, BT)]],
                    tbuf.at[nb], sem_g[nb])

            pltpu.make_async_copy(
                table_hbm.at[idx_v.at[pl.ds(0, BT)]],
                tbuf.at[n], sem_g[n]).wait()

            @pl.when(si >= NBUF)
            def _():
                _write_waits(n)

            @pl.when(si < N_TOKENS)
            def _():
                srow = jnp.full((LANES,), si, jnp.int32)

                @pl.loop(0, EMBED_DIM)
                def _(j):
                    scol = jnp.full((LANES,), j, jnp.int32)
                    v = plsc.load_gather(learned_v, [srow, scol])
                    for k in range(BT // LANES):
                        obuf[n, j, pl.ds(k * LANES, LANES)] = v

            @pl.when(si >= N_TOKENS)
            def _():
                @pl.loop(0, BT, unroll=2)
                def _(b):
                    bcol = jnp.full((LANES,), b, jnp.int32)
                    for c in range(JCH):
                        v = tbuf[n, b, pl.ds(c * LANES, LANES)]
                        plsc.store_scatter(obuf.at[n], [jrows[c], bcol], v)

            for jt in range(JT):
                pltpu.async_copy(
                    obuf.at[n, pl.ds(jt * 8, 8)],
                    out_hbm.at[si, jt, wid], sem_w[n])

    for n in range(NBUF):
        _write_waits(n)


def kernel(tokens, wte_weight, learned_embedding):
    tok32 = tokens.astype(jnp.int32)
    idx = tok32.T.reshape(SEQ, NUM_WORKERS, BT).transpose(1, 0, 2)
    idx = idx.reshape(NUM_WORKERS * SEQ * BT)
    out5 = _soft_embedding_sc(idx, wte_weight, learned_embedding)
    return out5.transpose(2, 4, 0, 1, 3).reshape(BATCH, SEQ, EMBED_DIM)


# speedup vs baseline: 1.2125x; 1.2125x over previous
import functools

import jax
import jax.numpy as jnp
from jax import lax
from jax.experimental import pallas as pl
from jax.experimental.pallas import tpu as pltpu
from jax.experimental.pallas import tpu_sc as plsc

BATCH = 4096
SEQ = 200
N_TOKENS = 10
EMBED_DIM = 64
LANES = 16

_SC_INFO = plsc.get_sparse_core_info()
NUM_WORKERS = _SC_INFO.num_cores * _SC_INFO.num_subcores
BT = BATCH // NUM_WORKERS
JT = EMBED_DIM // 8
NBUF = 2
JCH = EMBED_DIM // LANES


@functools.partial(
    pl.kernel,
    out_type=jax.ShapeDtypeStruct((SEQ, JT, NUM_WORKERS, 8, BT), jnp.float32),
    mesh=plsc.VectorSubcoreMesh(core_axis_name="c", subcore_axis_name="s"),
    scratch_types=[
        pltpu.VMEM((SEQ * BT,), jnp.int32),
        pltpu.VMEM((NBUF, BT, EMBED_DIM), jnp.float32),
        pltpu.VMEM((NBUF, EMBED_DIM, BT), jnp.float32),
        pltpu.VMEM((N_TOKENS, EMBED_DIM), jnp.float32),
        pltpu.SemaphoreType.DMA,
        [pltpu.SemaphoreType.DMA] * NBUF,
        [pltpu.SemaphoreType.DMA] * NBUF,
    ],
    compiler_params=pltpu.CompilerParams(use_tc_tiling_on_sc=False,
                                         needs_layout_passes=False),
)
def _soft_embedding_sc(idx_hbm, table_hbm, learned_hbm, out_hbm,
                       idx_v, tbuf, obuf, learned_v, sem_i, sem_g, sem_w):
    wid = lax.axis_index("s") * _SC_INFO.num_cores + lax.axis_index("c")

    pltpu.sync_copy(learned_hbm, learned_v)
    pltpu.async_copy(idx_hbm.at[pl.ds(wid * SEQ * BT, SEQ * BT)],
                     idx_v, sem_i)
    jrows = [lax.iota(jnp.int32, LANES) + (c * LANES) for c in range(JCH)]
    pltpu.make_async_copy(idx_hbm.at[pl.ds(0, SEQ * BT)], idx_v, sem_i).wait()

    pltpu.async_copy(table_hbm.at[idx_v.at[pl.ds(0, BT)]],
                     tbuf.at[0], sem_g[0])

    def _write_waits(n):
        for jt in range(JT):
            pltpu.make_async_copy(
                obuf.at[n, pl.ds(jt * 8, 8)],
                out_hbm.at[0, jt, 0], sem_w[n]).wait()

    @pl.loop(0, SEQ, step=NBUF)
    def _(s):
        for n in range(NBUF):
            si = s + n
            nb = (n + 1) % NBUF

            @pl.when(si + 1 < SEQ)
            def _():
                pltpu.async_copy(
                    table_hbm.at[idx_v.at[pl.ds((si + 1) * BT, BT)]],
                    tbuf.at[nb], sem_g[nb])

            pltpu.make_async_copy(
                table_hbm.at[idx_v.at[pl.ds(0, BT)]],
                tbuf.at[n], sem_g[n]).wait()

            @pl.when(si >= NBUF)
            def _():
                _write_waits(n)

            @pl.when(si < N_TOKENS)
            def _():
                srow = jnp.full((LANES,), si, jnp.int32)

                @pl.loop(0, EMBED_DIM)
                def _(j):
                    scol = jnp.full((LANES,), j, jnp.int32)
                    v = plsc.load_gather(learned_v, [srow, scol])
                    for k in range(BT // LANES):
                        obuf[n, j, pl.ds(k * LANES, LANES)] = v

            @pl.when(si >= N_TOKENS)
            def _():
                @plsc.parallel_loop(0, BT, unroll=4)
                def _(b):
                    bcol = jnp.full((LANES,), b, jnp.int32)
                    for c in range(JCH):
                        v = tbuf[n, b, pl.ds(c * LANES, LANES)]
                        plsc.store_scatter(obuf.at[n], [jrows[c], bcol], v)

            for jt in range(JT):
                pltpu.async_copy(
                    obuf.at[n, pl.ds(jt * 8, 8)],
                    out_hbm.at[si, jt, wid], sem_w[n])

    for n in range(NBUF):
        _write_waits(n)


def kernel(tokens, wte_weight, learned_embedding):
    tok32 = tokens.astype(jnp.int32)
    idx = tok32.T.reshape(SEQ, NUM_WORKERS, BT).transpose(1, 0, 2)
    idx = idx.reshape(NUM_WORKERS * SEQ * BT)
    out5 = _soft_embedding_sc(idx, wte_weight, learned_embedding)
    return out5.transpose(2, 4, 0, 1, 3).reshape(BATCH, SEQ, EMBED_DIM)
